# baseline (speedup 1.0000x reference)
import functools

import jax
import jax.numpy as jnp
from jax import lax
from jax.experimental import pallas as pl
from jax.experimental.pallas import tpu as pltpu
from jax.experimental.pallas import tpu_sc as plsc

B = 1024
T = 2048
D = 64
L = 16
NC = 2
NS = 16
NW = NC * NS
NTS = 16
TS = T // NTS
NBH = NW // NTS
BH = B // NBH
SB = 2
NPH = 4
BLK = SB * NPH
NBLK = BH // BLK


def _emb_body(idx_hbm, tok_hbm, pos_hbm, out_hbm, pos_v, idx_v,
              rb0, rb1, rb2, rb3,
              g0, g1, g2, g3, w0, w1, w2, w3, isem):
    rbufs = [rb0, rb1, rb2, rb3]
    gsems = [g0, g1, g2, g3]
    wsems = [w0, w1, w2, w3]

    wid = lax.axis_index("s") * NC + lax.axis_index("c")
    t0 = (wid % NTS) * TS
    bbase = (wid // NTS) * BH

    pltpu.sync_copy(pos_hbm.at[pl.ds(t0, TS)], pos_v)

    def gather_descs(blksel, ph, s):
        return [
            pltpu.make_async_copy(
                tok_hbm.at[idx_v.at[blksel, ph * SB + r]],
                rbufs[s].at[r], gsems[s])
            for r in range(SB)
        ]

    def out_slice(p):
        return out_hbm.at[pl.ds(bbase + p * SB, SB), pl.ds(t0, TS)]

    def add_pos(s):
        def row_body(r, rc):
            for j in range(SB):
                for k in range(D // L):
                    sl = pl.ds(k * L, L)
                    rbufs[s][j, r, sl] = rbufs[s][j, r, sl] + pos_v[r, sl]
            return rc
        lax.fori_loop(0, TS, row_body, 0)

    pltpu.sync_copy(idx_hbm.at[pl.ds(bbase, BLK), pl.ds(t0, TS)],
                    idx_v.at[0])
    for ph in (0, 1):
        for d in gather_descs(0, ph, ph):
            d.start()

    def block_body(i, carry):
        isel = i % 2
        nsel = (i + 1) % 2
        not_last = i < NBLK - 1

        for ph in range(NPH):
            p = i * NPH + ph
            s = ph
            s2 = (ph + 2) % 4

            if ph == 0:
                @pl.when(not_last)
                def _():
                    pltpu.async_copy(
                        idx_hbm.at[pl.ds(bbase + (i + 1) * BLK, BLK),
                                   pl.ds(t0, TS)],
                        idx_v.at[nsel], isem)

            if ph < 2:
                @pl.when(i > 0)
                def _():
                    pltpu.make_async_copy(
                        rbufs[s2], out_slice(p - 2), wsems[s2]).wait()

                for d in gather_descs(isel, ph + 2, s2):
                    d.start()
            else:
                pltpu.make_async_copy(
                    rbufs[s2], out_slice(p - 2), wsems[s2]).wait()
                if ph == 2:
                    @pl.when(not_last)
                    def _():
                        pltpu.make_async_copy(
                            idx_hbm.at[pl.ds(bbase + (i + 1) * BLK, BLK),
                                       pl.ds(t0, TS)],
                            idx_v.at[nsel], isem).wait()

                @pl.when(not_last)
                def _():
                    for d in gather_descs(nsel, ph - 2, s2):
                        d.start()

            for d in gather_descs(isel, ph, s):
                d.wait()
            add_pos(s)
            pltpu.async_copy(rbufs[s], out_slice(p), wsems[s])

        return carry

    lax.fori_loop(0, NBLK, block_body, 0)

    last = NPH * NBLK
    pltpu.make_async_copy(rbufs[2], out_slice(last - 2), wsems[2]).wait()
    pltpu.make_async_copy(rbufs[3], out_slice(last - 1), wsems[3]).wait()


@functools.partial(
    pl.kernel,
    out_type=jax.ShapeDtypeStruct((B, T, D), jnp.float32),
    mesh=plsc.VectorSubcoreMesh(core_axis_name="c", subcore_axis_name="s"),
    compiler_params=pltpu.CompilerParams(use_tc_tiling_on_sc=False),
    scratch_types=[
        pltpu.VMEM((TS, D), jnp.float32),
        pltpu.VMEM((2, BLK, TS), jnp.int32),
        pltpu.VMEM((SB, TS, D), jnp.float32),
        pltpu.VMEM((SB, TS, D), jnp.float32),
        pltpu.VMEM((SB, TS, D), jnp.float32),
        pltpu.VMEM((SB, TS, D), jnp.float32),
        pltpu.SemaphoreType.DMA,
        pltpu.SemaphoreType.DMA,
        pltpu.SemaphoreType.DMA,
        pltpu.SemaphoreType.DMA,
        pltpu.SemaphoreType.DMA,
        pltpu.SemaphoreType.DMA,
        pltpu.SemaphoreType.DMA,
        pltpu.SemaphoreType.DMA,
        pltpu.SemaphoreType.DMA,
    ],
)
def _emb_call(idx_hbm, tok_hbm, pos_hbm, out_hbm, pos_v, idx_v,
              rb0, rb1, rb2, rb3,
              g0, g1, g2, g3, w0, w1, w2, w3, isem):
    _emb_body(idx_hbm, tok_hbm, pos_hbm, out_hbm, pos_v, idx_v,
              rb0, rb1, rb2, rb3,
              g0, g1, g2, g3, w0, w1, w2, w3, isem)


def kernel(idx, tok_emb, pos_emb):
    return _emb_call(idx, tok_emb, pos_emb)
